# baseline (speedup 1.0000x reference)
import functools

import jax
import jax.numpy as jnp
from jax import lax
from jax.experimental import pallas as pl
from jax.experimental.pallas import tpu as pltpu
from jax.experimental.pallas import tpu_sc as plsc

N = 10000
E = 160000
D = 256
DH = D // 2
NSUB = 16
EPT = E // NSUB
CB = 16
NCHUNK = EPT // CB
NBUF = 8
NIDX = 16
NG = NCHUNK // NIDX
TAIL = NCHUNK - NG * NIDX
ZR = 200
NRC = N // ZR
MAXC = (NRC + NSUB - 1) // NSUB


def _sc_agg_body(x0, x1, er, zrows, zcnt, agg0, agg1, cntA, cntB,
                 sgidx, dgidx, ones_v, rows_v,
                 acc_sh, cnt_sh, sem_i, sem_c, *sems, with_cnt):
    sems_g = sems[:NBUF]
    sems_s = sems[NBUF:]
    cid = lax.axis_index("c")
    sid = lax.axis_index("s")

    pltpu.async_copy(er.at[0, sid, pl.ds(0, NIDX)], sgidx.at[0], sem_i)
    pltpu.async_copy(er.at[1, sid, pl.ds(0, NIDX)], dgidx.at[0], sem_i)

    for off in list(range(0, CB - 15, 16)) + [CB - 16]:
        ones_v[pl.ds(off, 16)] = jnp.ones((16,), jnp.float32)

    for j in range(MAXC):
        c = j * NSUB + sid

        @pl.when(c < NRC)
        def _(c=c, j=j):
            pltpu.async_copy(zrows.at[pl.ds(c * ZR, ZR)],
                             acc_sh.at[pl.ds(c * ZR, ZR)], sems_s[j % NBUF])
    for j in range(MAXC):
        c = j * NSUB + sid

        @pl.when(c < NRC)
        def _(c=c, j=j):
            pltpu.make_async_copy(zrows.at[pl.ds(c * ZR, ZR)],
                                  acc_sh.at[pl.ds(c * ZR, ZR)],
                                  sems_s[j % NBUF]).wait()

    if with_cnt:
        @pl.when(sid == 0)
        def _():
            pltpu.sync_copy(zcnt, cnt_sh)

    plsc.subcore_barrier()

    def drain_scatter(b, slot):
        pltpu.make_async_copy(rows_v.at[b], acc_sh.at[dgidx.at[slot, 0]],
                              sems_s[b]).wait()

    def do_pass(slot, j0, nbuf, first, cnt_core):
        for b in range(nbuf):
            if first is None:
                drain_scatter(b, slot)
            elif first is not True:
                @pl.when(first)
                def _(b=b):
                    drain_scatter(b, slot)

            @pl.when(cid == 0)
            def _(b=b):
                pltpu.async_copy(x0.at[sgidx.at[slot, j0 + b]],
                                 rows_v.at[b], sems_g[b])

            @pl.when(cid == 1)
            def _(b=b):
                pltpu.async_copy(x1.at[sgidx.at[slot, j0 + b]],
                                 rows_v.at[b], sems_g[b])

        for b in range(nbuf):
            pltpu.make_async_copy(x0.at[sgidx.at[slot, j0 + b]],
                                  rows_v.at[b], sems_g[b]).wait()
            pltpu.async_copy(rows_v.at[b], acc_sh.at[dgidx.at[slot, j0 + b]],
                             sems_s[b], add=True)
            if with_cnt:
                @pl.when(cid == cnt_core)
                def _(b=b):
                    pltpu.async_copy(ones_v, cnt_sh.at[dgidx.at[slot, j0 + b]],
                                     sem_c, add=True)

    def group(g, _):
        k0 = g * NIDX
        slot = lax.rem(g, 2)
        pltpu.make_async_copy(er.at[0, sid, pl.ds(k0, NIDX)],
                              sgidx.at[slot], sem_i).wait()
        pltpu.make_async_copy(er.at[1, sid, pl.ds(k0, NIDX)],
                              dgidx.at[slot], sem_i).wait()

        @pl.when(g + 1 < NG)
        def _():
            nslot = 1 - slot
            pltpu.async_copy(er.at[0, sid, pl.ds(k0 + NIDX, NIDX)],
                             sgidx.at[nslot], sem_i)
            pltpu.async_copy(er.at[1, sid, pl.ds(k0 + NIDX, NIDX)],
                             dgidx.at[nslot], sem_i)

        for h in range(NIDX // NBUF):
            do_pass(slot, h * NBUF, NBUF, (g > 0) if h == 0 else None, h % 2)
        if with_cnt:
            for _j in range(NIDX // 2):
                pltpu.make_async_copy(ones_v, cnt_sh.at[dgidx.at[slot, 0]],
                                      sem_c).wait()
        return _

    lax.fori_loop(0, NG, group, None)

    if TAIL:
        k0 = NG * NIDX
        pltpu.sync_copy(er.at[0, sid, pl.ds(k0, TAIL)],
                        sgidx.at[0, pl.ds(0, TAIL)])
        pltpu.sync_copy(er.at[1, sid, pl.ds(k0, TAIL)],
                        dgidx.at[0, pl.ds(0, TAIL)])
        j0 = 0
        while j0 < TAIL:
            nb = min(NBUF, TAIL - j0)
            do_pass(0, j0, nb, None, 0)
            j0 += nb
        if with_cnt:
            @pl.when(cid == 0)
            def _():
                for _j in range(TAIL):
                    pltpu.make_async_copy(ones_v, cnt_sh.at[dgidx.at[0, 0]],
                                          sem_c).wait()

    for b in range(NBUF):
        drain_scatter(b, 0)
    plsc.subcore_barrier()

    for j in range(MAXC):
        c = j * NSUB + sid

        @pl.when(jnp.logical_and(c < NRC, cid == 0))
        def _(c=c):
            pltpu.sync_copy(acc_sh.at[pl.ds(c * ZR, ZR)],
                            agg0.at[pl.ds(c * ZR, ZR)])

        @pl.when(jnp.logical_and(c < NRC, cid == 1))
        def _(c=c):
            pltpu.sync_copy(acc_sh.at[pl.ds(c * ZR, ZR)],
                            agg1.at[pl.ds(c * ZR, ZR)])

    if with_cnt:
        @pl.when(jnp.logical_and(cid == 0, sid == 0))
        def _():
            pltpu.sync_copy(cnt_sh, cntA)

        @pl.when(jnp.logical_and(cid == 1, sid == 0))
        def _():
            pltpu.sync_copy(cnt_sh, cntB)


def _make_sc_agg(with_cnt):
    mesh = plsc.VectorSubcoreMesh(core_axis_name="c", subcore_axis_name="s")
    out_type = [jax.ShapeDtypeStruct((N, DH), jnp.float32),
                jax.ShapeDtypeStruct((N, DH), jnp.float32)]
    if with_cnt:
        out_type.append(jax.ShapeDtypeStruct((N,), jnp.float32))
        out_type.append(jax.ShapeDtypeStruct((N,), jnp.float32))
    scratch = [
        pltpu.VMEM((2, NIDX, CB), jnp.int32),
        pltpu.VMEM((2, NIDX, CB), jnp.int32),
        pltpu.VMEM((CB,), jnp.float32),
        pltpu.VMEM((NBUF, CB, DH), jnp.float32),
        pltpu.VMEM_SHARED((N, DH), jnp.float32),
        pltpu.VMEM_SHARED((N,), jnp.float32),
        pltpu.SemaphoreType.DMA,
        pltpu.SemaphoreType.DMA,
    ] + [pltpu.SemaphoreType.DMA] * (2 * NBUF)
    body = functools.partial(_sc_agg_body, with_cnt=with_cnt)
    if with_cnt:
        def body_wc(x0, x1, er, zrows, zcnt, agg0, agg1, cA, cB, *s):
            body(x0, x1, er, zrows, zcnt, agg0, agg1, cA, cB, *s)
    else:
        def body_wc(x0, x1, er, zrows, zcnt, agg0, agg1, *s):
            body(x0, x1, er, zrows, zcnt, agg0, agg1, None, None, *s)
    return pl.kernel(body_wc, out_type=out_type, mesh=mesh,
                     scratch_types=scratch)


_sc_agg_cnt = _make_sc_agg(with_cnt=True)
_sc_agg = _make_sc_agg(with_cnt=False)


RB = 400
GRID = N // RB


def _dense1_body(a0, a1, cA, cB, xr, wl, bl, wr, h0, h1):
    deg = jnp.maximum(cA[...] + cB[...], 1.0)
    mean = jnp.concatenate([a0[...], a1[...]], axis=1) / deg
    res = (lax.dot_general(mean, wl[...], (((1,), (1,)), ((), ())),
                           preferred_element_type=jnp.float32)
           + bl[...]
           + lax.dot_general(xr[...], wr[...], (((1,), (1,)), ((), ())),
                             preferred_element_type=jnp.float32))
    h = jnp.maximum(res, 0.0)
    h0[...] = h[:, :DH]
    h1[...] = h[:, DH:]


def _dense2_body(a0, a1, cA, cB, h0, h1, wl, bl, wr, out):
    deg = jnp.maximum(cA[...] + cB[...], 1.0)
    mean = jnp.concatenate([a0[...], a1[...]], axis=1) / deg
    root = jnp.concatenate([h0[...], h1[...]], axis=1)
    out[...] = (lax.dot_general(mean, wl[...], (((1,), (1,)), ((), ())),
                                preferred_element_type=jnp.float32)
                + bl[...]
                + lax.dot_general(root, wr[...], (((1,), (1,)), ((), ())),
                                  preferred_element_type=jnp.float32))


def _row_spec(w):
    return pl.BlockSpec((RB, w), lambda i: (i, 0))


def _full_spec(shape):
    return pl.BlockSpec(shape, lambda i: tuple(0 for _ in shape))


_dense1 = pl.pallas_call(
    _dense1_body,
    grid=(GRID,),
    in_specs=[_row_spec(DH), _row_spec(DH), _row_spec(1), _row_spec(1),
              _row_spec(D),
              _full_spec((D, D)), _full_spec((1, D)), _full_spec((D, D))],
    out_specs=[_row_spec(DH), _row_spec(DH)],
    out_shape=[jax.ShapeDtypeStruct((N, DH), jnp.float32),
               jax.ShapeDtypeStruct((N, DH), jnp.float32)],
)

_dense2 = pl.pallas_call(
    _dense2_body,
    grid=(GRID,),
    in_specs=[_row_spec(DH), _row_spec(DH), _row_spec(1), _row_spec(1),
              _row_spec(DH), _row_spec(DH),
              _full_spec((D, D)), _full_spec((1, D)), _full_spec((D, D))],
    out_specs=_row_spec(D),
    out_shape=jax.ShapeDtypeStruct((N, D), jnp.float32),
)


def kernel(x, edge_index, W1l, b1l, W1r, W2l, b2l, W2r):
    er = edge_index.reshape(2, NSUB, NCHUNK, CB)
    x0 = x[:, :DH]
    x1 = x[:, DH:]
    zrows = jnp.zeros((N, DH), jnp.float32)
    zcnt = jnp.zeros((N,), jnp.float32)
    agg0, agg1, cntA, cntB = _sc_agg_cnt(x0, x1, er, zrows, zcnt)
    cA = cntA.reshape(N, 1)
    cB = cntB.reshape(N, 1)
    h0, h1 = _dense1(agg0, agg1, cA, cB, x, W1l, b1l.reshape(1, D), W1r)
    g0, g1 = _sc_agg(h0, h1, er, zrows, zcnt)
    out = _dense2(g0, g1, cA, cB, h0, h1, W2l, b2l.reshape(1, D), W2r)
    return out


# speedup vs baseline: 1.1338x; 1.1338x over previous
import functools

import jax
import jax.numpy as jnp
from jax import lax
from jax.experimental import pallas as pl
from jax.experimental.pallas import tpu as pltpu
from jax.experimental.pallas import tpu_sc as plsc

N = 10000
E = 160000
D = 256
DH = D // 2
NSUB = 16
EPT = E // NSUB
CB = 40
NCHUNK = EPT // CB
NBUF = 4
NIDX = 16
NG = NCHUNK // NIDX
TAIL = NCHUNK - NG * NIDX
ZR = 200
NRC = N // ZR
MAXC = (NRC + NSUB - 1) // NSUB


def _sc_agg_body(x0, x1, er, zrows, zcnt, agg0, agg1, cntA, cntB,
                 sgidx, dgidx, ones_v, rows_v,
                 acc_sh, cnt_sh, sem_i, sem_c, *sems, with_cnt):
    sems_g = sems[:NBUF]
    sems_s = sems[NBUF:]
    cid = lax.axis_index("c")
    sid = lax.axis_index("s")

    pltpu.async_copy(er.at[0, sid, pl.ds(0, NIDX)], sgidx.at[0], sem_i)
    pltpu.async_copy(er.at[1, sid, pl.ds(0, NIDX)], dgidx.at[0], sem_i)

    for off in list(range(0, CB - 15, 16)) + [CB - 16]:
        ones_v[pl.ds(off, 16)] = jnp.ones((16,), jnp.float32)

    for j in range(MAXC):
        c = j * NSUB + sid

        @pl.when(c < NRC)
        def _(c=c, j=j):
            pltpu.async_copy(zrows.at[pl.ds(c * ZR, ZR)],
                             acc_sh.at[pl.ds(c * ZR, ZR)], sems_s[j % NBUF])
    for j in range(MAXC):
        c = j * NSUB + sid

        @pl.when(c < NRC)
        def _(c=c, j=j):
            pltpu.make_async_copy(zrows.at[pl.ds(c * ZR, ZR)],
                                  acc_sh.at[pl.ds(c * ZR, ZR)],
                                  sems_s[j % NBUF]).wait()

    if with_cnt:
        @pl.when(sid == 0)
        def _():
            pltpu.sync_copy(zcnt, cnt_sh)

    plsc.subcore_barrier()

    def drain_scatter(b, slot):
        pltpu.make_async_copy(rows_v.at[b], acc_sh.at[dgidx.at[slot, 0]],
                              sems_s[b]).wait()

    def do_pass(slot, j0, nbuf, first, cnt_core):
        for b in range(nbuf):
            if first is None:
                drain_scatter(b, slot)
            elif first is not True:
                @pl.when(first)
                def _(b=b):
                    drain_scatter(b, slot)

            @pl.when(cid == 0)
            def _(b=b):
                pltpu.async_copy(x0.at[sgidx.at[slot, j0 + b]],
                                 rows_v.at[b], sems_g[b])

            @pl.when(cid == 1)
            def _(b=b):
                pltpu.async_copy(x1.at[sgidx.at[slot, j0 + b]],
                                 rows_v.at[b], sems_g[b])

        for b in range(nbuf):
            pltpu.make_async_copy(x0.at[sgidx.at[slot, j0 + b]],
                                  rows_v.at[b], sems_g[b]).wait()
            pltpu.async_copy(rows_v.at[b], acc_sh.at[dgidx.at[slot, j0 + b]],
                             sems_s[b], add=True)
            if with_cnt:
                @pl.when(cid == cnt_core)
                def _(b=b):
                    pltpu.async_copy(ones_v, cnt_sh.at[dgidx.at[slot, j0 + b]],
                                     sem_c, add=True)

    def group(g, _):
        k0 = g * NIDX
        slot = lax.rem(g, 2)
        pltpu.make_async_copy(er.at[0, sid, pl.ds(k0, NIDX)],
                              sgidx.at[slot], sem_i).wait()
        pltpu.make_async_copy(er.at[1, sid, pl.ds(k0, NIDX)],
                              dgidx.at[slot], sem_i).wait()

        @pl.when(g + 1 < NG)
        def _():
            nslot = 1 - slot
            pltpu.async_copy(er.at[0, sid, pl.ds(k0 + NIDX, NIDX)],
                             sgidx.at[nslot], sem_i)
            pltpu.async_copy(er.at[1, sid, pl.ds(k0 + NIDX, NIDX)],
                             dgidx.at[nslot], sem_i)

        for h in range(NIDX // NBUF):
            do_pass(slot, h * NBUF, NBUF, (g > 0) if h == 0 else None, h % 2)
        if with_cnt:
            for _j in range(NIDX // 2):
                pltpu.make_async_copy(ones_v, cnt_sh.at[dgidx.at[slot, 0]],
                                      sem_c).wait()
        return _

    lax.fori_loop(0, NG, group, None)

    if TAIL:
        k0 = NG * NIDX
        pltpu.sync_copy(er.at[0, sid, pl.ds(k0, TAIL)],
                        sgidx.at[0, pl.ds(0, TAIL)])
        pltpu.sync_copy(er.at[1, sid, pl.ds(k0, TAIL)],
                        dgidx.at[0, pl.ds(0, TAIL)])
        j0 = 0
        while j0 < TAIL:
            nb = min(NBUF, TAIL - j0)
            do_pass(0, j0, nb, None, 0)
            j0 += nb
        if with_cnt:
            @pl.when(cid == 0)
            def _():
                for _j in range(TAIL):
                    pltpu.make_async_copy(ones_v, cnt_sh.at[dgidx.at[0, 0]],
                                          sem_c).wait()

    for b in range(NBUF):
        drain_scatter(b, 0)
    plsc.subcore_barrier()

    for j in range(MAXC):
        c = j * NSUB + sid

        @pl.when(jnp.logical_and(c < NRC, cid == 0))
        def _(c=c):
            pltpu.sync_copy(acc_sh.at[pl.ds(c * ZR, ZR)],
                            agg0.at[pl.ds(c * ZR, ZR)])

        @pl.when(jnp.logical_and(c < NRC, cid == 1))
        def _(c=c):
            pltpu.sync_copy(acc_sh.at[pl.ds(c * ZR, ZR)],
                            agg1.at[pl.ds(c * ZR, ZR)])

    if with_cnt:
        @pl.when(jnp.logical_and(cid == 0, sid == 0))
        def _():
            pltpu.sync_copy(cnt_sh, cntA)

        @pl.when(jnp.logical_and(cid == 1, sid == 0))
        def _():
            pltpu.sync_copy(cnt_sh, cntB)


def _make_sc_agg(with_cnt):
    mesh = plsc.VectorSubcoreMesh(core_axis_name="c", subcore_axis_name="s")
    out_type = [jax.ShapeDtypeStruct((N, DH), jnp.float32),
                jax.ShapeDtypeStruct((N, DH), jnp.float32)]
    if with_cnt:
        out_type.append(jax.ShapeDtypeStruct((N,), jnp.float32))
        out_type.append(jax.ShapeDtypeStruct((N,), jnp.float32))
    scratch = [
        pltpu.VMEM((2, NIDX, CB), jnp.int32),
        pltpu.VMEM((2, NIDX, CB), jnp.int32),
        pltpu.VMEM((CB,), jnp.float32),
        pltpu.VMEM((NBUF, CB, DH), jnp.float32),
        pltpu.VMEM_SHARED((N, DH), jnp.float32),
        pltpu.VMEM_SHARED((N,), jnp.float32),
        pltpu.SemaphoreType.DMA,
        pltpu.SemaphoreType.DMA,
    ] + [pltpu.SemaphoreType.DMA] * (2 * NBUF)
    body = functools.partial(_sc_agg_body, with_cnt=with_cnt)
    if with_cnt:
        def body_wc(x0, x1, er, zrows, zcnt, agg0, agg1, cA, cB, *s):
            body(x0, x1, er, zrows, zcnt, agg0, agg1, cA, cB, *s)
    else:
        def body_wc(x0, x1, er, zrows, zcnt, agg0, agg1, *s):
            body(x0, x1, er, zrows, zcnt, agg0, agg1, None, None, *s)
    return pl.kernel(body_wc, out_type=out_type, mesh=mesh,
                     scratch_types=scratch)


_sc_agg_cnt = _make_sc_agg(with_cnt=True)
_sc_agg = _make_sc_agg(with_cnt=False)


RB = 400
GRID = N // RB


def _dense1_body(a0, a1, cA, cB, xr, wl, bl, wr, h0, h1):
    deg = jnp.maximum(cA[...] + cB[...], 1.0)
    mean = jnp.concatenate([a0[...], a1[...]], axis=1) / deg
    res = (lax.dot_general(mean, wl[...], (((1,), (1,)), ((), ())),
                           preferred_element_type=jnp.float32)
           + bl[...]
           + lax.dot_general(xr[...], wr[...], (((1,), (1,)), ((), ())),
                             preferred_element_type=jnp.float32))
    h = jnp.maximum(res, 0.0)
    h0[...] = h[:, :DH]
    h1[...] = h[:, DH:]


def _dense2_body(a0, a1, cA, cB, h0, h1, wl, bl, wr, out):
    deg = jnp.maximum(cA[...] + cB[...], 1.0)
    mean = jnp.concatenate([a0[...], a1[...]], axis=1) / deg
    root = jnp.concatenate([h0[...], h1[...]], axis=1)
    out[...] = (lax.dot_general(mean, wl[...], (((1,), (1,)), ((), ())),
                                preferred_element_type=jnp.float32)
                + bl[...]
                + lax.dot_general(root, wr[...], (((1,), (1,)), ((), ())),
                                  preferred_element_type=jnp.float32))


def _row_spec(w):
    return pl.BlockSpec((RB, w), lambda i: (i, 0))


def _full_spec(shape):
    return pl.BlockSpec(shape, lambda i: tuple(0 for _ in shape))


_dense1 = pl.pallas_call(
    _dense1_body,
    grid=(GRID,),
    in_specs=[_row_spec(DH), _row_spec(DH), _row_spec(1), _row_spec(1),
              _row_spec(D),
              _full_spec((D, D)), _full_spec((1, D)), _full_spec((D, D))],
    out_specs=[_row_spec(DH), _row_spec(DH)],
    out_shape=[jax.ShapeDtypeStruct((N, DH), jnp.float32),
               jax.ShapeDtypeStruct((N, DH), jnp.float32)],
)

_dense2 = pl.pallas_call(
    _dense2_body,
    grid=(GRID,),
    in_specs=[_row_spec(DH), _row_spec(DH), _row_spec(1), _row_spec(1),
              _row_spec(DH), _row_spec(DH),
              _full_spec((D, D)), _full_spec((1, D)), _full_spec((D, D))],
    out_specs=_row_spec(D),
    out_shape=jax.ShapeDtypeStruct((N, D), jnp.float32),
)


def kernel(x, edge_index, W1l, b1l, W1r, W2l, b2l, W2r):
    er = edge_index.reshape(2, NSUB, NCHUNK, CB)
    x0 = x[:, :DH]
    x1 = x[:, DH:]
    zrows = jnp.zeros((N, DH), jnp.float32)
    zcnt = jnp.zeros((N,), jnp.float32)
    agg0, agg1, cntA, cntB = _sc_agg_cnt(x0, x1, er, zrows, zcnt)
    cA = cntA.reshape(N, 1)
    cB = cntB.reshape(N, 1)
    h0, h1 = _dense1(agg0, agg1, cA, cB, x, W1l, b1l.reshape(1, D), W1r)
    g0, g1 = _sc_agg(h0, h1, er, zrows, zcnt)
    out = _dense2(g0, g1, cA, cB, h0, h1, W2l, b2l.reshape(1, D), W2r)
    return out
